# baseline (speedup 1.0000x reference)
import functools

import jax
import jax.numpy as jnp
from jax import lax
from jax.experimental import pallas as pl
from jax.experimental.pallas import tpu as pltpu
from jax.experimental.pallas import tpu_sc as plsc

_NC = 2
_NS = 16
_NW = _NC * _NS


def _head_body(x_ref, w1_ref, b1_ref, w2_ref, b2_ref, o_ref):
    h = jnp.dot(x_ref[...], w1_ref[...], preferred_element_type=jnp.float32)
    h = jnp.maximum(h + b1_ref[...], 0.0)
    h = jnp.dot(h, w2_ref[...], preferred_element_type=jnp.float32)
    o_ref[...] = jnp.maximum(h + b2_ref[...], 0.0)


def _head(x, W1, b1, W2, b2, rb):
    n, d = x.shape
    h = W1.shape[1]
    nb = n // rb
    return pl.pallas_call(
        _head_body,
        grid=(nb,),
        in_specs=[
            pl.BlockSpec((rb, d), lambda i: (i, 0)),
            pl.BlockSpec((d, h), lambda i: (0, 0)),
            pl.BlockSpec((1, h), lambda i: (0, 0)),
            pl.BlockSpec((h, h), lambda i: (0, 0)),
            pl.BlockSpec((1, h), lambda i: (0, 0)),
        ],
        out_specs=pl.BlockSpec((rb, h), lambda i: (i, 0)),
        out_shape=jax.ShapeDtypeStruct((n, h), jnp.float32),
    )(x, W1, b1.reshape(1, h), W2, b2.reshape(1, h))


def _sc_scatter(f2h0, f2h1, srcp, dstp, n, h, ept, ch):
    n_acc, hh = f2h0.shape
    nch = ept // ch
    jrows = ch // 128
    npair = nch // 2
    zrows = n_acc // _NS

    mesh = plsc.VectorSubcoreMesh(core_axis_name="c", subcore_axis_name="s")

    @functools.partial(
        pl.kernel,
        mesh=mesh,
        compiler_params=pltpu.CompilerParams(use_tc_tiling_on_sc=False),
        out_type=jax.ShapeDtypeStruct((_NC, 2, n_acc, hh), jnp.float32),
        scratch_types=[
            [pltpu.VMEM((jrows, 128), jnp.int32) for _ in range(4)],
            [pltpu.VMEM((jrows, 128), jnp.int32) for _ in range(4)],
            [pltpu.VMEM((ch, hh), jnp.float32) for _ in range(4)],
            pltpu.VMEM_SHARED((n_acc, hh), jnp.float32),
            pltpu.VMEM_SHARED((n_acc, hh), jnp.float32),
            [pltpu.SemaphoreType.DMA for _ in range(4)],
            pltpu.SemaphoreType.DMA,
        ],
    )
    def body(f0_hbm, f1_hbm, src_hbm, dst_hbm, out_hbm, srcv, dstv, rowsv,
             acc_sh, tab_sh, gsem, ssem):
        c = lax.axis_index("c")
        s = lax.axis_index("s")
        wid = c * _NS + s
        base = wid * (ept // 128)

        def fire(ci, b, f_hbm):
            table = f_hbm if b < 2 else tab_sh
            r0 = base + ci * jrows
            pltpu.sync_copy(src_hbm.at[pl.ds(r0, jrows)], srcv[b])
            pltpu.sync_copy(dst_hbm.at[pl.ds(r0, jrows)], dstv[b])
            for j in range(jrows):
                pltpu.async_copy(table.at[srcv[b].at[j]],
                                 rowsv[b].at[pl.ds(j * 128, 128)], gsem[b])

        def drain_scatter(b):
            pltpu.make_async_copy(f0_hbm.at[pl.ds(0, ch)], rowsv[b],
                                  gsem[b]).wait()
            cps = [
                pltpu.async_copy(rowsv[b].at[pl.ds(j * 128, 128)],
                                 acc_sh.at[dstv[b].at[j]], ssem, add=True)
                for j in range(jrows)
            ]
            for cp in cps:
                cp.wait()

        for half, f_hbm in ((0, f0_hbm), (1, f1_hbm)):
            pltpu.sync_copy(f_hbm.at[pl.ds(s * zrows, zrows)],
                            acc_sh.at[pl.ds(s * zrows, zrows)])
            pltpu.sync_copy(f_hbm.at[pl.ds(s * zrows, zrows)],
                            tab_sh.at[pl.ds(s * zrows, zrows)])
            plsc.subcore_barrier()
            for b in range(4):
                fire(b, b, f_hbm)

            def ring(g, carry):
                for b in range(4):
                    drain_scatter(b)
                    fire(4 * g + b + 4, b, f_hbm)
                return carry

            lax.fori_loop(0, nch // 4 - 1, ring, 0)
            for b in range(4):
                drain_scatter(b)
            plsc.subcore_barrier()
            pltpu.sync_copy(acc_sh.at[pl.ds(s * zrows, zrows)],
                            out_hbm.at[c, half, pl.ds(s * zrows, zrows)])

    o = body(f2h0, f2h1, srcp, dstp)
    return jnp.concatenate([o[:, 0], o[:, 1]], axis=-1)[:, :n, :]


def _tail_body(batch_ref, f2_ref, a0_ref, a1_ref, wg_ref, bg_ref, wl_ref,
               bl_ref, wt_ref, bt_ref, wa_ref, ba_ref, wb_ref, bb_ref,
               wc_ref, bc_ref, wk_ref, bk_ref, out_ref, pooled_ref,
               h_scr, gp_scr, mx_scr, num_scr, den_scr, *, rb, nb, g_seg):
    p = pl.program_id(0)
    i = pl.program_id(1)
    bids = batch_ref[0, 0, :]
    oh = bids[:, None] == lax.broadcasted_iota(jnp.int32, (rb, g_seg), 1)

    @pl.when(p == 0)
    def _pass0():
        f = a0_ref[...] + a1_ref[...] - f2_ref[...]
        f = jnp.dot(f, wg_ref[...], preferred_element_type=jnp.float32)
        f = jnp.maximum(f + bg_ref[...], 0.0)
        f = jnp.dot(f, wl_ref[...], preferred_element_type=jnp.float32) + bl_ref[...]
        h = jnp.dot(f, wt_ref[...], preferred_element_type=jnp.float32) + bt_ref[...]
        ta = jnp.tanh(jnp.dot(h, wa_ref[...], preferred_element_type=jnp.float32)
                      + ba_ref[...])
        sb = jax.nn.sigmoid(jnp.dot(h, wb_ref[...],
                                    preferred_element_type=jnp.float32)
                            + bb_ref[...])
        gp = jnp.dot(ta * sb, wc_ref[...], preferred_element_type=jnp.float32) \
            + bc_ref[...]
        h_scr[pl.ds(i * rb, rb), :] = h
        gp_scr[pl.ds(i * rb, rb), :] = gp

        @pl.when(i == 0)
        def _():
            mx_scr[...] = jnp.full_like(mx_scr[...], -1e30)

        for g in range(g_seg):
            m = jnp.max(jnp.where(oh[:, g:g + 1], gp, -1e30), axis=0)
            mx_scr[g, :] = jnp.maximum(mx_scr[g, :], m)

    @pl.when(p == 1)
    def _pass1():
        @pl.when(i == 0)
        def _():
            num_scr[...] = jnp.zeros_like(num_scr[...])
            den_scr[...] = jnp.zeros_like(den_scr[...])

        ohf = oh.astype(jnp.float32)
        h = h_scr[pl.ds(i * rb, rb), :]
        gp = gp_scr[pl.ds(i * rb, rb), :]
        smax = jnp.dot(ohf, mx_scr[...], preferred_element_type=jnp.float32)
        e = jnp.exp(gp - smax)
        dims = (((0,), (0,)), ((), ()))
        den_scr[...] += lax.dot_general(ohf, e, dims,
                                        preferred_element_type=jnp.float32)
        num_scr[...] += lax.dot_general(ohf, h * e, dims,
                                        preferred_element_type=jnp.float32)

        @pl.when(i == nb - 1)
        def _():
            pooled = num_scr[...] / (den_scr[...] + 1e-16)
            pooled_ref[...] = pooled
            out_ref[...] = jnp.dot(pooled, wk_ref[...],
                                   preferred_element_type=jnp.float32) \
                + bk_ref[...]


def _tail(batch3, f2, a0, a1, W_gin, b_gin, W_lin, b_lin, W_tail, b_tail,
          Wa, ba, Wb, bb, Wc, bc, W_cls, b_cls, rb, g_seg):
    n, h = f2.shape
    c = W_tail.shape[1]
    t = W_cls.shape[1]
    nb = n // rb

    def rowmap(p, i):
        return (i * (1 - p), 0)

    wspec = lambda shp: pl.BlockSpec(shp, lambda p, i: tuple(0 for _ in shp))
    body = functools.partial(_tail_body, rb=rb, nb=nb, g_seg=g_seg)
    return pl.pallas_call(
        body,
        grid=(2, nb),
        in_specs=[
            pl.BlockSpec((1, 1, rb), lambda p, i: (i, 0, 0)),
            pl.BlockSpec((rb, h), rowmap),
            pl.BlockSpec((rb, h), rowmap),
            pl.BlockSpec((rb, h), rowmap),
            wspec((h, h)), wspec((1, h)),
            wspec((h, h)), wspec((1, h)),
            wspec((h, c)), wspec((1, c)),
            wspec((c, 64)), wspec((1, 64)),
            wspec((c, 64)), wspec((1, 64)),
            wspec((64, c)), wspec((1, c)),
            wspec((c, t)), wspec((1, t)),
        ],
        out_specs=[
            pl.BlockSpec((g_seg, t), lambda p, i: (0, 0)),
            pl.BlockSpec((g_seg, c), lambda p, i: (0, 0)),
        ],
        out_shape=[
            jax.ShapeDtypeStruct((g_seg, t), jnp.float32),
            jax.ShapeDtypeStruct((g_seg, c), jnp.float32),
        ],
        scratch_shapes=[
            pltpu.VMEM((n, c), jnp.float32),
            pltpu.VMEM((n, c), jnp.float32),
            pltpu.VMEM((g_seg, c), jnp.float32),
            pltpu.VMEM((g_seg, c), jnp.float32),
            pltpu.VMEM((g_seg, c), jnp.float32),
        ],
    )(batch3, f2, a0, a1,
      W_gin, b_gin.reshape(1, h), W_lin, b_lin.reshape(1, h),
      W_tail, b_tail.reshape(1, c), Wa, ba.reshape(1, 64),
      Wb, bb.reshape(1, 64), Wc, bc.reshape(1, c),
      W_cls, b_cls.reshape(1, t))


def kernel(x, edge_index, batch, W_head, b_head, W_chead, b_chead,
           W_gin, b_gin, W_lin, b_lin, W_tail, b_tail,
           Wa, ba, Wb, bb, Wc, bc, W_cls, b_cls):
    n = x.shape[0]
    e = edge_index.shape[1]
    h = W_head.shape[1]
    g_seg = 8
    rb = 2000

    f2 = _head(x, W_head, b_head, W_chead, b_chead, rb)

    ch = 512
    ept = -(-(-(-e // _NW)) // ch) * ch
    e_pad = ept * _NW
    n_acc = -(-(n + 1) // 128) * 128
    src = jnp.concatenate(
        [edge_index[0], jnp.zeros((e_pad - e,), jnp.int32)]).reshape(-1, 128)
    dst = jnp.concatenate(
        [edge_index[1], jnp.full((e_pad - e,), n, jnp.int32)]).reshape(-1, 128)
    f2p = jnp.concatenate([f2, jnp.zeros((n_acc - n, h), jnp.float32)])
    parts = _sc_scatter(f2p[:, :h // 2], f2p[:, h // 2:], src, dst,
                        n, h, ept, ch)

    batch3 = batch.reshape(n // rb, 1, rb)
    out, pooled = _tail(batch3, f2, parts[0], parts[1],
                        W_gin, b_gin, W_lin, b_lin, W_tail, b_tail,
                        Wa, ba, Wb, bb, Wc, bc, W_cls, b_cls, rb, g_seg)
    return (out, pooled)


# speedup vs baseline: 1.1205x; 1.1205x over previous
import functools

import jax
import jax.numpy as jnp
from jax import lax
from jax.experimental import pallas as pl
from jax.experimental.pallas import tpu as pltpu
from jax.experimental.pallas import tpu_sc as plsc

_NC = 2
_NS = 16
_NW = _NC * _NS


def _head_body(x_ref, w1_ref, b1_ref, w2_ref, b2_ref, o_ref):
    h = jnp.dot(x_ref[...], w1_ref[...], preferred_element_type=jnp.float32)
    h = jnp.maximum(h + b1_ref[...], 0.0)
    h = jnp.dot(h, w2_ref[...], preferred_element_type=jnp.float32)
    o_ref[...] = jnp.maximum(h + b2_ref[...], 0.0)


def _head(x, W1, b1, W2, b2, rb):
    n, d = x.shape
    h = W1.shape[1]
    nb = n // rb
    return pl.pallas_call(
        _head_body,
        grid=(nb,),
        in_specs=[
            pl.BlockSpec((rb, d), lambda i: (i, 0)),
            pl.BlockSpec((d, h), lambda i: (0, 0)),
            pl.BlockSpec((1, h), lambda i: (0, 0)),
            pl.BlockSpec((h, h), lambda i: (0, 0)),
            pl.BlockSpec((1, h), lambda i: (0, 0)),
        ],
        out_specs=pl.BlockSpec((rb, h), lambda i: (i, 0)),
        out_shape=jax.ShapeDtypeStruct((n, h), jnp.float32),
    )(x, W1, b1.reshape(1, h), W2, b2.reshape(1, h))


def _sc_scatter(f2h0, f2h1, srcp, dstp, n, h, ept, ch):
    n_acc, hh = f2h0.shape
    nch = ept // ch
    jrows = ch // 128
    npair = nch // 2
    zrows = n_acc // _NS

    mesh = plsc.VectorSubcoreMesh(core_axis_name="c", subcore_axis_name="s")

    @functools.partial(
        pl.kernel,
        mesh=mesh,
        compiler_params=pltpu.CompilerParams(use_tc_tiling_on_sc=False),
        out_type=jax.ShapeDtypeStruct((_NC, 2, n_acc, hh), jnp.float32),
        scratch_types=[
            [pltpu.VMEM((jrows, 128), jnp.int32) for _ in range(4)],
            [pltpu.VMEM((jrows, 128), jnp.int32) for _ in range(4)],
            [pltpu.VMEM((ch, hh), jnp.float32) for _ in range(4)],
            pltpu.VMEM_SHARED((n_acc, hh), jnp.float32),
            pltpu.VMEM_SHARED((n_acc, hh), jnp.float32),
            [pltpu.SemaphoreType.DMA for _ in range(4)],
            pltpu.SemaphoreType.DMA,
        ],
    )
    def body(f0_hbm, f1_hbm, src_hbm, dst_hbm, out_hbm, srcv, dstv, rowsv,
             acc_sh, tab_sh, gsem, ssem):
        c = lax.axis_index("c")
        s = lax.axis_index("s")
        wid = c * _NS + s
        base = wid * (ept // 128)

        def fire(ci, b, f_hbm):
            table = tab_sh
            r0 = base + ci * jrows
            pltpu.sync_copy(src_hbm.at[pl.ds(r0, jrows)], srcv[b])
            pltpu.sync_copy(dst_hbm.at[pl.ds(r0, jrows)], dstv[b])
            for j in range(jrows):
                pltpu.async_copy(table.at[srcv[b].at[j]],
                                 rowsv[b].at[pl.ds(j * 128, 128)], gsem[b])

        def drain_scatter(b):
            pltpu.make_async_copy(f0_hbm.at[pl.ds(0, ch)], rowsv[b],
                                  gsem[b]).wait()
            cps = [
                pltpu.async_copy(rowsv[b].at[pl.ds(j * 128, 128)],
                                 acc_sh.at[dstv[b].at[j]], ssem, add=True)
                for j in range(jrows)
            ]
            for cp in cps:
                cp.wait()

        for half, f_hbm in ((0, f0_hbm), (1, f1_hbm)):
            pltpu.sync_copy(f_hbm.at[pl.ds(s * zrows, zrows)],
                            acc_sh.at[pl.ds(s * zrows, zrows)])
            pltpu.sync_copy(f_hbm.at[pl.ds(s * zrows, zrows)],
                            tab_sh.at[pl.ds(s * zrows, zrows)])
            plsc.subcore_barrier()
            for b in range(4):
                fire(b, b, f_hbm)

            def ring(g, carry):
                for b in range(4):
                    drain_scatter(b)
                    fire(4 * g + b + 4, b, f_hbm)
                return carry

            lax.fori_loop(0, nch // 4 - 1, ring, 0)
            for b in range(4):
                drain_scatter(b)
            plsc.subcore_barrier()
            pltpu.sync_copy(acc_sh.at[pl.ds(s * zrows, zrows)],
                            out_hbm.at[c, half, pl.ds(s * zrows, zrows)])

    o = body(f2h0, f2h1, srcp, dstp)
    return jnp.concatenate([o[:, 0], o[:, 1]], axis=-1)[:, :n, :]


def _tail_body(batch_ref, f2_ref, a0_ref, a1_ref, wg_ref, bg_ref, wl_ref,
               bl_ref, wt_ref, bt_ref, wa_ref, ba_ref, wb_ref, bb_ref,
               wc_ref, bc_ref, wk_ref, bk_ref, out_ref, pooled_ref,
               h_scr, gp_scr, mx_scr, num_scr, den_scr, *, rb, nb, g_seg):
    p = pl.program_id(0)
    i = pl.program_id(1)
    bids = batch_ref[0, 0, :]
    oh = bids[:, None] == lax.broadcasted_iota(jnp.int32, (rb, g_seg), 1)

    @pl.when(p == 0)
    def _pass0():
        f = a0_ref[...] + a1_ref[...] - f2_ref[...]
        f = jnp.dot(f, wg_ref[...], preferred_element_type=jnp.float32)
        f = jnp.maximum(f + bg_ref[...], 0.0)
        f = jnp.dot(f, wl_ref[...], preferred_element_type=jnp.float32) + bl_ref[...]
        h = jnp.dot(f, wt_ref[...], preferred_element_type=jnp.float32) + bt_ref[...]
        ta = jnp.tanh(jnp.dot(h, wa_ref[...], preferred_element_type=jnp.float32)
                      + ba_ref[...])
        sb = jax.nn.sigmoid(jnp.dot(h, wb_ref[...],
                                    preferred_element_type=jnp.float32)
                            + bb_ref[...])
        gp = jnp.dot(ta * sb, wc_ref[...], preferred_element_type=jnp.float32) \
            + bc_ref[...]
        h_scr[pl.ds(i * rb, rb), :] = h
        gp_scr[pl.ds(i * rb, rb), :] = gp

        @pl.when(i == 0)
        def _():
            mx_scr[...] = jnp.full_like(mx_scr[...], -1e30)

        for g in range(g_seg):
            m = jnp.max(jnp.where(oh[:, g:g + 1], gp, -1e30), axis=0)
            mx_scr[g, :] = jnp.maximum(mx_scr[g, :], m)

    @pl.when(p == 1)
    def _pass1():
        @pl.when(i == 0)
        def _():
            num_scr[...] = jnp.zeros_like(num_scr[...])
            den_scr[...] = jnp.zeros_like(den_scr[...])

        ohf = oh.astype(jnp.float32)
        h = h_scr[pl.ds(i * rb, rb), :]
        gp = gp_scr[pl.ds(i * rb, rb), :]
        smax = jnp.dot(ohf, mx_scr[...], preferred_element_type=jnp.float32)
        e = jnp.exp(gp - smax)
        dims = (((0,), (0,)), ((), ()))
        den_scr[...] += lax.dot_general(ohf, e, dims,
                                        preferred_element_type=jnp.float32)
        num_scr[...] += lax.dot_general(ohf, h * e, dims,
                                        preferred_element_type=jnp.float32)

        @pl.when(i == nb - 1)
        def _():
            pooled = num_scr[...] / (den_scr[...] + 1e-16)
            pooled_ref[...] = pooled
            out_ref[...] = jnp.dot(pooled, wk_ref[...],
                                   preferred_element_type=jnp.float32) \
                + bk_ref[...]


def _tail(batch3, f2, a0, a1, W_gin, b_gin, W_lin, b_lin, W_tail, b_tail,
          Wa, ba, Wb, bb, Wc, bc, W_cls, b_cls, rb, g_seg):
    n, h = f2.shape
    c = W_tail.shape[1]
    t = W_cls.shape[1]
    nb = n // rb

    def rowmap(p, i):
        return (i * (1 - p), 0)

    wspec = lambda shp: pl.BlockSpec(shp, lambda p, i: tuple(0 for _ in shp))
    body = functools.partial(_tail_body, rb=rb, nb=nb, g_seg=g_seg)
    return pl.pallas_call(
        body,
        grid=(2, nb),
        in_specs=[
            pl.BlockSpec((1, 1, rb), lambda p, i: (i, 0, 0)),
            pl.BlockSpec((rb, h), rowmap),
            pl.BlockSpec((rb, h), rowmap),
            pl.BlockSpec((rb, h), rowmap),
            wspec((h, h)), wspec((1, h)),
            wspec((h, h)), wspec((1, h)),
            wspec((h, c)), wspec((1, c)),
            wspec((c, 64)), wspec((1, 64)),
            wspec((c, 64)), wspec((1, 64)),
            wspec((64, c)), wspec((1, c)),
            wspec((c, t)), wspec((1, t)),
        ],
        out_specs=[
            pl.BlockSpec((g_seg, t), lambda p, i: (0, 0)),
            pl.BlockSpec((g_seg, c), lambda p, i: (0, 0)),
        ],
        out_shape=[
            jax.ShapeDtypeStruct((g_seg, t), jnp.float32),
            jax.ShapeDtypeStruct((g_seg, c), jnp.float32),
        ],
        scratch_shapes=[
            pltpu.VMEM((n, c), jnp.float32),
            pltpu.VMEM((n, c), jnp.float32),
            pltpu.VMEM((g_seg, c), jnp.float32),
            pltpu.VMEM((g_seg, c), jnp.float32),
            pltpu.VMEM((g_seg, c), jnp.float32),
        ],
    )(batch3, f2, a0, a1,
      W_gin, b_gin.reshape(1, h), W_lin, b_lin.reshape(1, h),
      W_tail, b_tail.reshape(1, c), Wa, ba.reshape(1, 64),
      Wb, bb.reshape(1, 64), Wc, bc.reshape(1, c),
      W_cls, b_cls.reshape(1, t))


def kernel(x, edge_index, batch, W_head, b_head, W_chead, b_chead,
           W_gin, b_gin, W_lin, b_lin, W_tail, b_tail,
           Wa, ba, Wb, bb, Wc, bc, W_cls, b_cls):
    n = x.shape[0]
    e = edge_index.shape[1]
    h = W_head.shape[1]
    g_seg = 8
    rb = 2000

    f2 = _head(x, W_head, b_head, W_chead, b_chead, rb)

    ch = 512
    ept = -(-(-(-e // _NW)) // ch) * ch
    e_pad = ept * _NW
    n_acc = -(-(n + 1) // 128) * 128
    src = jnp.concatenate(
        [edge_index[0], jnp.zeros((e_pad - e,), jnp.int32)]).reshape(-1, 128)
    dst = jnp.concatenate(
        [edge_index[1], jnp.full((e_pad - e,), n, jnp.int32)]).reshape(-1, 128)
    f2p = jnp.concatenate([f2, jnp.zeros((n_acc - n, h), jnp.float32)])
    parts = _sc_scatter(f2p[:, :h // 2], f2p[:, h // 2:], src, dst,
                        n, h, ept, ch)

    batch3 = batch.reshape(n // rb, 1, rb)
    out, pooled = _tail(batch3, f2, parts[0], parts[1],
                        W_gin, b_gin, W_lin, b_lin, W_tail, b_tail,
                        Wa, ba, Wb, bb, Wc, bc, W_cls, b_cls, rb, g_seg)
    return (out, pooled)


# speedup vs baseline: 1.2114x; 1.0810x over previous
import functools

import jax
import jax.numpy as jnp
from jax import lax
from jax.experimental import pallas as pl
from jax.experimental.pallas import tpu as pltpu
from jax.experimental.pallas import tpu_sc as plsc

_NC = 2
_NS = 16
_NW = _NC * _NS


def _head_body(x_ref, w1_ref, b1_ref, w2_ref, b2_ref, o_ref):
    h = jnp.dot(x_ref[...], w1_ref[...], preferred_element_type=jnp.float32)
    h = jnp.maximum(h + b1_ref[...], 0.0)
    h = jnp.dot(h, w2_ref[...], preferred_element_type=jnp.float32)
    o_ref[...] = jnp.maximum(h + b2_ref[...], 0.0)


def _head(x, W1, b1, W2, b2, rb):
    n, d = x.shape
    h = W1.shape[1]
    nb = n // rb
    return pl.pallas_call(
        _head_body,
        grid=(nb,),
        in_specs=[
            pl.BlockSpec((rb, d), lambda i: (i, 0)),
            pl.BlockSpec((d, h), lambda i: (0, 0)),
            pl.BlockSpec((1, h), lambda i: (0, 0)),
            pl.BlockSpec((h, h), lambda i: (0, 0)),
            pl.BlockSpec((1, h), lambda i: (0, 0)),
        ],
        out_specs=pl.BlockSpec((rb, h), lambda i: (i, 0)),
        out_shape=jax.ShapeDtypeStruct((n, h), jnp.float32),
    )(x, W1, b1.reshape(1, h), W2, b2.reshape(1, h))


def _sc_scatter(f2h0, f2h1, srcp, dstp, n, h, ept, ch):
    n_acc, hh = f2h0.shape
    nch = ept // ch
    jrows = ch // 128
    npair = nch // 2
    zrows = n_acc // _NS

    mesh = plsc.VectorSubcoreMesh(core_axis_name="c", subcore_axis_name="s")

    @functools.partial(
        pl.kernel,
        mesh=mesh,
        compiler_params=pltpu.CompilerParams(use_tc_tiling_on_sc=False),
        out_type=jax.ShapeDtypeStruct((_NC, 2, n_acc, hh), jnp.float32),
        scratch_types=[
            [pltpu.VMEM((jrows, 128), jnp.int32) for _ in range(2)],
            [pltpu.VMEM((jrows, 128), jnp.int32) for _ in range(2)],
            [pltpu.VMEM((ch, hh), jnp.float32) for _ in range(2)],
            pltpu.VMEM_SHARED((n_acc, hh), jnp.float32),
            pltpu.VMEM_SHARED((n_acc, hh), jnp.float32),
            [pltpu.SemaphoreType.DMA for _ in range(2)],
            pltpu.SemaphoreType.DMA,
        ],
    )
    def body(f0_hbm, f1_hbm, src_hbm, dst_hbm, out_hbm, srcv, dstv, rowsv,
             acc_sh, tab_sh, gsem, ssem):
        c = lax.axis_index("c")
        s = lax.axis_index("s")
        wid = c * _NS + s
        base = wid * (ept // 128)

        def fire(ci, b, f_hbm):
            table = tab_sh
            r0 = base + ci * jrows
            pltpu.sync_copy(src_hbm.at[pl.ds(r0, jrows)], srcv[b])
            pltpu.sync_copy(dst_hbm.at[pl.ds(r0, jrows)], dstv[b])
            for j in range(jrows):
                pltpu.async_copy(table.at[srcv[b].at[j]],
                                 rowsv[b].at[pl.ds(j * 128, 128)], gsem[b])

        def drain_scatter(b):
            pltpu.make_async_copy(f0_hbm.at[pl.ds(0, ch)], rowsv[b],
                                  gsem[b]).wait()
            cps = [
                pltpu.async_copy(rowsv[b].at[pl.ds(j * 128, 128)],
                                 acc_sh.at[dstv[b].at[j]], ssem, add=True)
                for j in range(jrows)
            ]
            for cp in cps:
                cp.wait()

        for half, f_hbm in ((0, f0_hbm), (1, f1_hbm)):
            pltpu.sync_copy(f_hbm.at[pl.ds(s * zrows, zrows)],
                            acc_sh.at[pl.ds(s * zrows, zrows)])
            pltpu.sync_copy(f_hbm.at[pl.ds(s * zrows, zrows)],
                            tab_sh.at[pl.ds(s * zrows, zrows)])
            plsc.subcore_barrier()
            for b in range(2):
                fire(b, b, f_hbm)

            def ring(g, carry):
                for b in range(2):
                    drain_scatter(b)
                    fire(2 * g + b + 2, b, f_hbm)
                return carry

            lax.fori_loop(0, nch // 2 - 1, ring, 0)
            for b in range(2):
                drain_scatter(b)
            plsc.subcore_barrier()
            pltpu.sync_copy(acc_sh.at[pl.ds(s * zrows, zrows)],
                            out_hbm.at[c, half, pl.ds(s * zrows, zrows)])

    o = body(f2h0, f2h1, srcp, dstp)
    return jnp.concatenate([o[:, 0], o[:, 1]], axis=-1)[:, :n, :]


def _tail_body(batch_ref, f2_ref, a0_ref, a1_ref, wg_ref, bg_ref, wl_ref,
               bl_ref, wt_ref, bt_ref, wa_ref, ba_ref, wb_ref, bb_ref,
               wc_ref, bc_ref, wk_ref, bk_ref, out_ref, pooled_ref,
               h_scr, gp_scr, mx_scr, num_scr, den_scr, *, rb, nb, g_seg):
    p = pl.program_id(0)
    i = pl.program_id(1)
    bids = batch_ref[0, 0, :]
    oh = bids[:, None] == lax.broadcasted_iota(jnp.int32, (rb, g_seg), 1)

    @pl.when(p == 0)
    def _pass0():
        f = a0_ref[...] + a1_ref[...] - f2_ref[...]
        f = jnp.dot(f, wg_ref[...], preferred_element_type=jnp.float32)
        f = jnp.maximum(f + bg_ref[...], 0.0)
        f = jnp.dot(f, wl_ref[...], preferred_element_type=jnp.float32) + bl_ref[...]
        h = jnp.dot(f, wt_ref[...], preferred_element_type=jnp.float32) + bt_ref[...]
        ta = jnp.tanh(jnp.dot(h, wa_ref[...], preferred_element_type=jnp.float32)
                      + ba_ref[...])
        sb = jax.nn.sigmoid(jnp.dot(h, wb_ref[...],
                                    preferred_element_type=jnp.float32)
                            + bb_ref[...])
        gp = jnp.dot(ta * sb, wc_ref[...], preferred_element_type=jnp.float32) \
            + bc_ref[...]
        h_scr[pl.ds(i * rb, rb), :] = h
        gp_scr[pl.ds(i * rb, rb), :] = gp

        @pl.when(i == 0)
        def _():
            mx_scr[...] = jnp.full_like(mx_scr[...], -1e30)

        for g in range(g_seg):
            m = jnp.max(jnp.where(oh[:, g:g + 1], gp, -1e30), axis=0)
            mx_scr[g, :] = jnp.maximum(mx_scr[g, :], m)

    @pl.when(p == 1)
    def _pass1():
        @pl.when(i == 0)
        def _():
            num_scr[...] = jnp.zeros_like(num_scr[...])
            den_scr[...] = jnp.zeros_like(den_scr[...])

        ohf = oh.astype(jnp.float32)
        h = h_scr[pl.ds(i * rb, rb), :]
        gp = gp_scr[pl.ds(i * rb, rb), :]
        smax = jnp.dot(ohf, mx_scr[...], preferred_element_type=jnp.float32)
        e = jnp.exp(gp - smax)
        dims = (((0,), (0,)), ((), ()))
        den_scr[...] += lax.dot_general(ohf, e, dims,
                                        preferred_element_type=jnp.float32)
        num_scr[...] += lax.dot_general(ohf, h * e, dims,
                                        preferred_element_type=jnp.float32)

        @pl.when(i == nb - 1)
        def _():
            pooled = num_scr[...] / (den_scr[...] + 1e-16)
            pooled_ref[...] = pooled
            out_ref[...] = jnp.dot(pooled, wk_ref[...],
                                   preferred_element_type=jnp.float32) \
                + bk_ref[...]


def _tail(batch3, f2, a0, a1, W_gin, b_gin, W_lin, b_lin, W_tail, b_tail,
          Wa, ba, Wb, bb, Wc, bc, W_cls, b_cls, rb, g_seg):
    n, h = f2.shape
    c = W_tail.shape[1]
    t = W_cls.shape[1]
    nb = n // rb

    def rowmap(p, i):
        return (i * (1 - p), 0)

    wspec = lambda shp: pl.BlockSpec(shp, lambda p, i: tuple(0 for _ in shp))
    body = functools.partial(_tail_body, rb=rb, nb=nb, g_seg=g_seg)
    return pl.pallas_call(
        body,
        grid=(2, nb),
        in_specs=[
            pl.BlockSpec((1, 1, rb), lambda p, i: (i, 0, 0)),
            pl.BlockSpec((rb, h), rowmap),
            pl.BlockSpec((rb, h), rowmap),
            pl.BlockSpec((rb, h), rowmap),
            wspec((h, h)), wspec((1, h)),
            wspec((h, h)), wspec((1, h)),
            wspec((h, c)), wspec((1, c)),
            wspec((c, 64)), wspec((1, 64)),
            wspec((c, 64)), wspec((1, 64)),
            wspec((64, c)), wspec((1, c)),
            wspec((c, t)), wspec((1, t)),
        ],
        out_specs=[
            pl.BlockSpec((g_seg, t), lambda p, i: (0, 0)),
            pl.BlockSpec((g_seg, c), lambda p, i: (0, 0)),
        ],
        out_shape=[
            jax.ShapeDtypeStruct((g_seg, t), jnp.float32),
            jax.ShapeDtypeStruct((g_seg, c), jnp.float32),
        ],
        scratch_shapes=[
            pltpu.VMEM((n, c), jnp.float32),
            pltpu.VMEM((n, c), jnp.float32),
            pltpu.VMEM((g_seg, c), jnp.float32),
            pltpu.VMEM((g_seg, c), jnp.float32),
            pltpu.VMEM((g_seg, c), jnp.float32),
        ],
    )(batch3, f2, a0, a1,
      W_gin, b_gin.reshape(1, h), W_lin, b_lin.reshape(1, h),
      W_tail, b_tail.reshape(1, c), Wa, ba.reshape(1, 64),
      Wb, bb.reshape(1, 64), Wc, bc.reshape(1, c),
      W_cls, b_cls.reshape(1, t))


def kernel(x, edge_index, batch, W_head, b_head, W_chead, b_chead,
           W_gin, b_gin, W_lin, b_lin, W_tail, b_tail,
           Wa, ba, Wb, bb, Wc, bc, W_cls, b_cls):
    n = x.shape[0]
    e = edge_index.shape[1]
    h = W_head.shape[1]
    g_seg = 8
    rb = 2000

    f2 = _head(x, W_head, b_head, W_chead, b_chead, rb)

    ch = 1024
    ept = -(-(-(-e // _NW)) // ch) * ch
    e_pad = ept * _NW
    n_acc = -(-(n + 1) // 128) * 128
    src = jnp.concatenate(
        [edge_index[0], jnp.zeros((e_pad - e,), jnp.int32)]).reshape(-1, 128)
    dst = jnp.concatenate(
        [edge_index[1], jnp.full((e_pad - e,), n, jnp.int32)]).reshape(-1, 128)
    f2p = jnp.concatenate([f2, jnp.zeros((n_acc - n, h), jnp.float32)])
    parts = _sc_scatter(f2p[:, :h // 2], f2p[:, h // 2:], src, dst,
                        n, h, ept, ch)

    batch3 = batch.reshape(n // rb, 1, rb)
    out, pooled = _tail(batch3, f2, parts[0], parts[1],
                        W_gin, b_gin, W_lin, b_lin, W_tail, b_tail,
                        Wa, ba, Wb, bb, Wc, bc, W_cls, b_cls, rb, g_seg)
    return (out, pooled)


# speedup vs baseline: 1.3445x; 1.1099x over previous
import functools

import jax
import jax.numpy as jnp
from jax import lax
from jax.experimental import pallas as pl
from jax.experimental.pallas import tpu as pltpu
from jax.experimental.pallas import tpu_sc as plsc

_NC = 2
_NS = 16
_NW = _NC * _NS


def _head_body(x_ref, w1_ref, b1_ref, w2_ref, b2_ref, o_ref):
    h = jnp.dot(x_ref[...], w1_ref[...], preferred_element_type=jnp.float32)
    h = jnp.maximum(h + b1_ref[...], 0.0)
    h = jnp.dot(h, w2_ref[...], preferred_element_type=jnp.float32)
    o_ref[...] = jnp.maximum(h + b2_ref[...], 0.0)


def _head(x, W1, b1, W2, b2, rb, n_out):
    n, d = x.shape
    h = W1.shape[1]
    nb = n // rb
    return pl.pallas_call(
        _head_body,
        grid=(nb,),
        in_specs=[
            pl.BlockSpec((rb, d), lambda i: (i, 0)),
            pl.BlockSpec((d, h), lambda i: (0, 0)),
            pl.BlockSpec((1, h), lambda i: (0, 0)),
            pl.BlockSpec((h, h), lambda i: (0, 0)),
            pl.BlockSpec((1, h), lambda i: (0, 0)),
        ],
        out_specs=pl.BlockSpec((rb, h), lambda i: (i, 0)),
        out_shape=jax.ShapeDtypeStruct((n_out, h), jnp.float32),
    )(x, W1, b1.reshape(1, h), W2, b2.reshape(1, h))


def _sc_scatter(f2, srcp, dstp, n, h, ept, ch):
    n_acc = f2.shape[0]
    hh = h // 2
    nch = ept // ch
    jrows = ch // 128
    npair = nch // 2
    zrows = n_acc // _NS

    mesh = plsc.VectorSubcoreMesh(core_axis_name="c", subcore_axis_name="s")

    @functools.partial(
        pl.kernel,
        mesh=mesh,
        compiler_params=pltpu.CompilerParams(use_tc_tiling_on_sc=False),
        out_type=jax.ShapeDtypeStruct((_NC, n_acc, h), jnp.float32),
        scratch_types=[
            [pltpu.VMEM((jrows, 128), jnp.int32) for _ in range(2)],
            [pltpu.VMEM((jrows, 128), jnp.int32) for _ in range(2)],
            [pltpu.VMEM((ch, hh), jnp.float32) for _ in range(2)],
            pltpu.VMEM_SHARED((n_acc, hh), jnp.float32),
            pltpu.VMEM_SHARED((n_acc, hh), jnp.float32),
            [pltpu.SemaphoreType.DMA for _ in range(2)],
            pltpu.SemaphoreType.DMA,
        ],
    )
    def body(f_hbm, src_hbm, dst_hbm, out_hbm, srcv, dstv, rowsv,
             acc_sh, tab_sh, gsem, ssem):
        c = lax.axis_index("c")
        s = lax.axis_index("s")
        wid = c * _NS + s
        base = wid * (ept // 128)

        def fire(ci, b):
            r0 = base + ci * jrows
            pltpu.sync_copy(src_hbm.at[pl.ds(r0, jrows)], srcv[b])
            pltpu.sync_copy(dst_hbm.at[pl.ds(r0, jrows)], dstv[b])
            for j in range(jrows):
                pltpu.async_copy(tab_sh.at[srcv[b].at[j]],
                                 rowsv[b].at[pl.ds(j * 128, 128)], gsem[b])

        def drain_scatter(b):
            pltpu.make_async_copy(f_hbm.at[pl.ds(0, ch), pl.ds(0, hh)],
                                  rowsv[b], gsem[b]).wait()
            cps = [
                pltpu.async_copy(rowsv[b].at[pl.ds(j * 128, 128)],
                                 acc_sh.at[dstv[b].at[j]], ssem, add=True)
                for j in range(jrows)
            ]
            for cp in cps:
                cp.wait()

        for half in (0, 1):
            cols = pl.ds(half * hh, hh)
            pltpu.sync_copy(f_hbm.at[pl.ds(s * zrows, zrows), cols],
                            acc_sh.at[pl.ds(s * zrows, zrows)])
            pltpu.sync_copy(f_hbm.at[pl.ds(s * zrows, zrows), cols],
                            tab_sh.at[pl.ds(s * zrows, zrows)])
            plsc.subcore_barrier()
            for b in range(2):
                fire(b, b)

            def ring(g, carry):
                for b in range(2):
                    drain_scatter(b)
                    fire(2 * g + b + 2, b)
                return carry

            lax.fori_loop(0, nch // 2 - 1, ring, 0)
            for b in range(2):
                drain_scatter(b)
            plsc.subcore_barrier()
            pltpu.sync_copy(acc_sh.at[pl.ds(s * zrows, zrows)],
                            out_hbm.at[c, pl.ds(s * zrows, zrows), cols])

    return body(f2, srcp, dstp)


def _tail_body(batch_ref, f2_ref, a0_ref, a1_ref, wg_ref, bg_ref, wl_ref,
               bl_ref, wt_ref, bt_ref, wa_ref, ba_ref, wb_ref, bb_ref,
               wc_ref, bc_ref, wk_ref, bk_ref, out_ref, pooled_ref,
               h_scr, gp_scr, mx_scr, num_scr, den_scr, *, rb, nb, g_seg):
    p = pl.program_id(0)
    i = pl.program_id(1)
    bids = batch_ref[0, 0, :]
    oh = bids[:, None] == lax.broadcasted_iota(jnp.int32, (rb, g_seg), 1)

    @pl.when(p == 0)
    def _pass0():
        f = a0_ref[0] + a1_ref[0] - f2_ref[...]
        f = jnp.dot(f, wg_ref[...], preferred_element_type=jnp.float32)
        f = jnp.maximum(f + bg_ref[...], 0.0)
        f = jnp.dot(f, wl_ref[...], preferred_element_type=jnp.float32) + bl_ref[...]
        h = jnp.dot(f, wt_ref[...], preferred_element_type=jnp.float32) + bt_ref[...]
        ta = jnp.tanh(jnp.dot(h, wa_ref[...], preferred_element_type=jnp.float32)
                      + ba_ref[...])
        sb = jax.nn.sigmoid(jnp.dot(h, wb_ref[...],
                                    preferred_element_type=jnp.float32)
                            + bb_ref[...])
        gp = jnp.dot(ta * sb, wc_ref[...], preferred_element_type=jnp.float32) \
            + bc_ref[...]
        h_scr[pl.ds(i * rb, rb), :] = h
        gp_scr[pl.ds(i * rb, rb), :] = gp

        @pl.when(i == 0)
        def _():
            mx_scr[...] = jnp.full_like(mx_scr[...], -1e30)

        for g in range(g_seg):
            m = jnp.max(jnp.where(oh[:, g:g + 1], gp, -1e30), axis=0)
            mx_scr[g, :] = jnp.maximum(mx_scr[g, :], m)

    @pl.when(p == 1)
    def _pass1():
        @pl.when(i == 0)
        def _():
            num_scr[...] = jnp.zeros_like(num_scr[...])
            den_scr[...] = jnp.zeros_like(den_scr[...])

        ohf = oh.astype(jnp.float32)
        h = h_scr[pl.ds(i * rb, rb), :]
        gp = gp_scr[pl.ds(i * rb, rb), :]
        smax = jnp.dot(ohf, mx_scr[...], preferred_element_type=jnp.float32)
        e = jnp.exp(gp - smax)
        dims = (((0,), (0,)), ((), ()))
        den_scr[...] += lax.dot_general(ohf, e, dims,
                                        preferred_element_type=jnp.float32)
        num_scr[...] += lax.dot_general(ohf, h * e, dims,
                                        preferred_element_type=jnp.float32)

        @pl.when(i == nb - 1)
        def _():
            pooled = num_scr[...] / (den_scr[...] + 1e-16)
            pooled_ref[...] = pooled
            out_ref[...] = jnp.dot(pooled, wk_ref[...],
                                   preferred_element_type=jnp.float32) \
                + bk_ref[...]


def _tail(batch3, f2, parts, n, W_gin, b_gin, W_lin, b_lin, W_tail, b_tail,
          Wa, ba, Wb, bb, Wc, bc, W_cls, b_cls, rb, g_seg):
    h = f2.shape[1]
    c = W_tail.shape[1]
    t = W_cls.shape[1]
    nb = n // rb

    def rowmap(p, i):
        return (i * (1 - p), 0)

    wspec = lambda shp: pl.BlockSpec(shp, lambda p, i: tuple(0 for _ in shp))
    body = functools.partial(_tail_body, rb=rb, nb=nb, g_seg=g_seg)
    return pl.pallas_call(
        body,
        grid=(2, nb),
        in_specs=[
            pl.BlockSpec((1, 1, rb), lambda p, i: (i, 0, 0)),
            pl.BlockSpec((rb, h), rowmap),
            pl.BlockSpec((1, rb, h), lambda p, i: (0, i * (1 - p), 0)),
            pl.BlockSpec((1, rb, h), lambda p, i: (1, i * (1 - p), 0)),
            wspec((h, h)), wspec((1, h)),
            wspec((h, h)), wspec((1, h)),
            wspec((h, c)), wspec((1, c)),
            wspec((c, 64)), wspec((1, 64)),
            wspec((c, 64)), wspec((1, 64)),
            wspec((64, c)), wspec((1, c)),
            wspec((c, t)), wspec((1, t)),
        ],
        out_specs=[
            pl.BlockSpec((g_seg, t), lambda p, i: (0, 0)),
            pl.BlockSpec((g_seg, c), lambda p, i: (0, 0)),
        ],
        out_shape=[
            jax.ShapeDtypeStruct((g_seg, t), jnp.float32),
            jax.ShapeDtypeStruct((g_seg, c), jnp.float32),
        ],
        scratch_shapes=[
            pltpu.VMEM((n, c), jnp.float32),
            pltpu.VMEM((n, c), jnp.float32),
            pltpu.VMEM((g_seg, c), jnp.float32),
            pltpu.VMEM((g_seg, c), jnp.float32),
            pltpu.VMEM((g_seg, c), jnp.float32),
        ],
    )(batch3, f2, parts, parts,
      W_gin, b_gin.reshape(1, h), W_lin, b_lin.reshape(1, h),
      W_tail, b_tail.reshape(1, c), Wa, ba.reshape(1, 64),
      Wb, bb.reshape(1, 64), Wc, bc.reshape(1, c),
      W_cls, b_cls.reshape(1, t))


def kernel(x, edge_index, batch, W_head, b_head, W_chead, b_chead,
           W_gin, b_gin, W_lin, b_lin, W_tail, b_tail,
           Wa, ba, Wb, bb, Wc, bc, W_cls, b_cls):
    n = x.shape[0]
    e = edge_index.shape[1]
    h = W_head.shape[1]
    g_seg = 8
    rb = 2000

    ch = 1024
    ept = -(-(-(-e // _NW)) // ch) * ch
    e_pad = ept * _NW
    n_acc = -(-(n + 1) // 128) * 128
    src = jnp.concatenate(
        [edge_index[0], jnp.zeros((e_pad - e,), jnp.int32)]).reshape(-1, 128)
    dst = jnp.concatenate(
        [edge_index[1], jnp.full((e_pad - e,), n, jnp.int32)]).reshape(-1, 128)

    f2 = _head(x, W_head, b_head, W_chead, b_chead, rb, n_acc)
    parts = _sc_scatter(f2, src, dst, n, h, ept, ch)

    batch3 = batch.reshape(n // rb, 1, rb)
    out, pooled = _tail(batch3, f2, parts, n,
                        W_gin, b_gin, W_lin, b_lin, W_tail, b_tail,
                        Wa, ba, Wb, bb, Wc, bc, W_cls, b_cls, rb, g_seg)
    return (out, pooled)


# speedup vs baseline: 1.3754x; 1.0231x over previous
import functools

import jax
import jax.numpy as jnp
from jax import lax
from jax.experimental import pallas as pl
from jax.experimental.pallas import tpu as pltpu
from jax.experimental.pallas import tpu_sc as plsc

_NC = 2
_NS = 16
_NW = _NC * _NS


def _head_body(x_ref, w1_ref, b1_ref, w2_ref, b2_ref, o_ref):
    h = jnp.dot(x_ref[...], w1_ref[...], preferred_element_type=jnp.float32)
    h = jnp.maximum(h + b1_ref[...], 0.0)
    h = jnp.dot(h, w2_ref[...], preferred_element_type=jnp.float32)
    o_ref[...] = jnp.maximum(h + b2_ref[...], 0.0)


def _head(x, W1, b1, W2, b2, rb, n_out):
    n, d = x.shape
    h = W1.shape[1]
    nb = n // rb
    return pl.pallas_call(
        _head_body,
        grid=(nb,),
        in_specs=[
            pl.BlockSpec((rb, d), lambda i: (i, 0)),
            pl.BlockSpec((d, h), lambda i: (0, 0)),
            pl.BlockSpec((1, h), lambda i: (0, 0)),
            pl.BlockSpec((h, h), lambda i: (0, 0)),
            pl.BlockSpec((1, h), lambda i: (0, 0)),
        ],
        out_specs=pl.BlockSpec((rb, h), lambda i: (i, 0)),
        out_shape=jax.ShapeDtypeStruct((n_out, h), jnp.float32),
    )(x, W1, b1.reshape(1, h), W2, b2.reshape(1, h))


def _sc_scatter(f2, srcp, dstp, n, h, ept, ch):
    n_acc = f2.shape[0]
    hh = h // 2
    nch = ept // ch
    jrows = ch // 128
    npair = nch // 2
    zrows = n_acc // _NS

    mesh = plsc.VectorSubcoreMesh(core_axis_name="c", subcore_axis_name="s")

    @functools.partial(
        pl.kernel,
        mesh=mesh,
        compiler_params=pltpu.CompilerParams(use_tc_tiling_on_sc=False),
        out_type=jax.ShapeDtypeStruct((_NC, n_acc, h), jnp.float32),
        scratch_types=[
            [pltpu.VMEM((jrows, 128), jnp.int32) for _ in range(2)],
            [pltpu.VMEM((jrows, 128), jnp.int32) for _ in range(2)],
            [pltpu.VMEM((ch, hh), jnp.float32) for _ in range(2)],
            pltpu.VMEM_SHARED((n_acc, hh), jnp.float32),
            pltpu.VMEM_SHARED((n_acc, hh), jnp.float32),
            [pltpu.SemaphoreType.DMA for _ in range(2)],
            pltpu.SemaphoreType.DMA,
        ],
    )
    def body(f_hbm, src_hbm, dst_hbm, out_hbm, srcv, dstv, rowsv,
             acc_sh, tab_sh, gsem, ssem):
        c = lax.axis_index("c")
        s = lax.axis_index("s")
        wid = c * _NS + s
        base = wid * (ept // 128)

        def fire(ci, b):
            r0 = base + ci * jrows
            pltpu.sync_copy(src_hbm.at[pl.ds(r0, jrows)], srcv[b])
            pltpu.sync_copy(dst_hbm.at[pl.ds(r0, jrows)], dstv[b])
            for j in range(jrows):
                pltpu.async_copy(tab_sh.at[srcv[b].at[j]],
                                 rowsv[b].at[pl.ds(j * 128, 128)], gsem[b])

        def drain_scatter(b):
            pltpu.make_async_copy(f_hbm.at[pl.ds(0, ch), pl.ds(0, hh)],
                                  rowsv[b], gsem[b]).wait()
            cps = [
                pltpu.async_copy(rowsv[b].at[pl.ds(j * 128, 128)],
                                 acc_sh.at[dstv[b].at[j]], ssem, add=True)
                for j in range(jrows)
            ]
            for cp in cps:
                cp.wait()

        for half in (0, 1):
            cols = pl.ds(half * hh, hh)
            pltpu.sync_copy(f_hbm.at[pl.ds(s * zrows, zrows), cols],
                            acc_sh.at[pl.ds(s * zrows, zrows)])
            pltpu.sync_copy(f_hbm.at[pl.ds(s * zrows, zrows), cols],
                            tab_sh.at[pl.ds(s * zrows, zrows)])
            plsc.subcore_barrier()
            for b in range(2):
                fire(b, b)

            def ring(g, carry):
                for b in range(2):
                    drain_scatter(b)
                    fire(2 * g + b + 2, b)
                return carry

            lax.fori_loop(0, nch // 2 - 1, ring, 0)
            for b in range(2):
                drain_scatter(b)
            plsc.subcore_barrier()
            pltpu.sync_copy(acc_sh.at[pl.ds(s * zrows, zrows)],
                            out_hbm.at[c, pl.ds(s * zrows, zrows), cols])

    return body(f2, srcp, dstp)


def _tail_body(batch_ref, f2_ref, a0_ref, a1_ref, wg_ref, bg_ref, wl_ref,
               bl_ref, wt_ref, bt_ref, wa_ref, ba_ref, wb_ref, bb_ref,
               wc_ref, bc_ref, wk_ref, bk_ref, out_ref, pooled_ref,
               h_scr, gp_scr, mx_scr, num_scr, den_scr, *, rb, nb, g_seg):
    p = pl.program_id(0)
    i = pl.program_id(1)
    bids = batch_ref[0, 0, :]
    oh = bids[:, None] == lax.broadcasted_iota(jnp.int32, (rb, g_seg), 1)

    @pl.when(p == 0)
    def _pass0():
        f = a0_ref[0] + a1_ref[0] - f2_ref[...]
        f = jnp.dot(f, wg_ref[...], preferred_element_type=jnp.float32)
        f = jnp.maximum(f + bg_ref[...], 0.0)
        f = jnp.dot(f, wl_ref[...], preferred_element_type=jnp.float32) + bl_ref[...]
        h = jnp.dot(f, wt_ref[...], preferred_element_type=jnp.float32) + bt_ref[...]
        ta = jnp.tanh(jnp.dot(h, wa_ref[...], preferred_element_type=jnp.float32)
                      + ba_ref[...])
        sb = jax.nn.sigmoid(jnp.dot(h, wb_ref[...],
                                    preferred_element_type=jnp.float32)
                            + bb_ref[...])
        gp = jnp.dot(ta * sb, wc_ref[...], preferred_element_type=jnp.float32) \
            + bc_ref[...]
        h_scr[pl.ds(i * rb, rb), :] = h
        gp_scr[pl.ds(i * rb, rb), :] = gp

        @pl.when(i == 0)
        def _():
            mx_scr[...] = jnp.full_like(mx_scr[...], -1e30)

        for g in range(g_seg):
            m = jnp.max(jnp.where(oh[:, g:g + 1], gp, -1e30), axis=0)
            mx_scr[g, :] = jnp.maximum(mx_scr[g, :], m)

    @pl.when(p == 1)
    def _pass1():
        @pl.when(i == 0)
        def _():
            num_scr[...] = jnp.zeros_like(num_scr[...])
            den_scr[...] = jnp.zeros_like(den_scr[...])

        ohf = oh.astype(jnp.float32)
        h = h_scr[pl.ds(i * rb, rb), :]
        gp = gp_scr[pl.ds(i * rb, rb), :]
        smax = jnp.dot(ohf, mx_scr[...], preferred_element_type=jnp.float32)
        e = jnp.exp(gp - smax)
        dims = (((0,), (0,)), ((), ()))
        den_scr[...] += lax.dot_general(ohf, e, dims,
                                        preferred_element_type=jnp.float32)
        num_scr[...] += lax.dot_general(ohf, h * e, dims,
                                        preferred_element_type=jnp.float32)

        @pl.when(i == nb - 1)
        def _():
            pooled = num_scr[...] / (den_scr[...] + 1e-16)
            pooled_ref[...] = pooled
            out_ref[...] = jnp.dot(pooled, wk_ref[...],
                                   preferred_element_type=jnp.float32) \
                + bk_ref[...]


def _tail(batch3, f2, parts, n, W_gin, b_gin, W_lin, b_lin, W_tail, b_tail,
          Wa, ba, Wb, bb, Wc, bc, W_cls, b_cls, rb, g_seg):
    h = f2.shape[1]
    c = W_tail.shape[1]
    t = W_cls.shape[1]
    nb = n // rb

    def rowmap(p, i):
        return (i * (1 - p), 0)

    wspec = lambda shp: pl.BlockSpec(shp, lambda p, i: tuple(0 for _ in shp))
    body = functools.partial(_tail_body, rb=rb, nb=nb, g_seg=g_seg)
    return pl.pallas_call(
        body,
        grid=(2, nb),
        in_specs=[
            pl.BlockSpec((1, 1, rb), lambda p, i: (i, 0, 0)),
            pl.BlockSpec((rb, h), rowmap),
            pl.BlockSpec((1, rb, h), lambda p, i: (0, i * (1 - p), 0)),
            pl.BlockSpec((1, rb, h), lambda p, i: (1, i * (1 - p), 0)),
            wspec((h, h)), wspec((1, h)),
            wspec((h, h)), wspec((1, h)),
            wspec((h, c)), wspec((1, c)),
            wspec((c, 64)), wspec((1, 64)),
            wspec((c, 64)), wspec((1, 64)),
            wspec((64, c)), wspec((1, c)),
            wspec((c, t)), wspec((1, t)),
        ],
        out_specs=[
            pl.BlockSpec((g_seg, t), lambda p, i: (0, 0)),
            pl.BlockSpec((g_seg, c), lambda p, i: (0, 0)),
        ],
        out_shape=[
            jax.ShapeDtypeStruct((g_seg, t), jnp.float32),
            jax.ShapeDtypeStruct((g_seg, c), jnp.float32),
        ],
        scratch_shapes=[
            pltpu.VMEM((n, c), jnp.float32),
            pltpu.VMEM((n, c), jnp.float32),
            pltpu.VMEM((g_seg, c), jnp.float32),
            pltpu.VMEM((g_seg, c), jnp.float32),
            pltpu.VMEM((g_seg, c), jnp.float32),
        ],
    )(batch3, f2, parts, parts,
      W_gin, b_gin.reshape(1, h), W_lin, b_lin.reshape(1, h),
      W_tail, b_tail.reshape(1, c), Wa, ba.reshape(1, 64),
      Wb, bb.reshape(1, 64), Wc, bc.reshape(1, c),
      W_cls, b_cls.reshape(1, t))


def kernel(x, edge_index, batch, W_head, b_head, W_chead, b_chead,
           W_gin, b_gin, W_lin, b_lin, W_tail, b_tail,
           Wa, ba, Wb, bb, Wc, bc, W_cls, b_cls):
    n = x.shape[0]
    e = edge_index.shape[1]
    h = W_head.shape[1]
    g_seg = 8
    rb = 2000

    ch = 1280
    ept = -(-(-(-e // _NW)) // (2 * ch)) * (2 * ch)
    e_pad = ept * _NW
    n_acc = -(-(n + 1) // 128) * 128
    src = jnp.concatenate(
        [edge_index[0], jnp.zeros((e_pad - e,), jnp.int32)]).reshape(-1, 128)
    dst = jnp.concatenate(
        [edge_index[1], jnp.full((e_pad - e,), n, jnp.int32)]).reshape(-1, 128)

    f2 = _head(x, W_head, b_head, W_chead, b_chead, rb, n_acc)
    parts = _sc_scatter(f2, src, dst, n, h, ept, ch)

    batch3 = batch.reshape(n // rb, 1, rb)
    out, pooled = _tail(batch3, f2, parts, n,
                        W_gin, b_gin, W_lin, b_lin, W_tail, b_tail,
                        Wa, ba, Wb, bb, Wc, bc, W_cls, b_cls, rb, g_seg)
    return (out, pooled)
